# baseline (speedup 1.0000x reference)
import functools

import jax
import jax.numpy as jnp
from jax import lax
from jax.experimental import pallas as pl
from jax.experimental.pallas import tpu as pltpu

_N = 8192
_K = 20
_R = 256
_LANES = 128
_HI = _N // _LANES


def _knn_feature_kernel(x_ref, x3t_ref, xp_ref, out_ref, d_ref, feat_ref):
    q = x3t_ref[0]
    keys = x_ref[0, :3, :]
    inner = jnp.dot(q, keys, preferred_element_type=jnp.float32)
    inner_m2 = -2.0 * inner
    xxq = jnp.sum(q * q, axis=1, keepdims=True)
    xxk = jnp.sum(keys * keys, axis=0, keepdims=True)
    d_ref[...] = ((-xxq) - inner_m2) - xxk

    iota = lax.broadcasted_iota(jnp.int32, (_R, _N), 1)
    iota_hi = lax.broadcasted_iota(jnp.int32, (_R, _HI), 1)
    iota_lo = lax.broadcasted_iota(jnp.int32, (_R, _LANES), 1)
    xp = xp_ref[0]

    m = jnp.full((_R, 1), jnp.float32(jnp.inf))
    fi = jnp.full((_R, 1), jnp.int32(-1))
    for j in range(_K):
        d = d_ref[...]
        el = (d < m) | ((d == m) & (iota > fi))
        m_n = jnp.max(jnp.where(el, d, jnp.float32(-1e30)), axis=1,
                      keepdims=True)
        c = jnp.where((d == m_n) & ((m_n < m) | (iota > fi)), iota,
                      jnp.int32(_N))
        fi = jnp.min(c, axis=1, keepdims=True)
        m = m_n
        hi = fi // _LANES
        lo = fi - hi * _LANES
        ohhi = (iota_hi == hi).astype(jnp.float32)
        t2 = jax.lax.dot(ohhi, xp, precision=jax.lax.Precision.HIGHEST,
                         preferred_element_type=jnp.float32)
        ohlo = (iota_lo == lo).astype(jnp.float32)
        for c in range(6):
            sl = t2[:, c * _LANES:(c + 1) * _LANES]
            feat_ref[c, :, j] = jnp.sum(sl * ohlo, axis=1)

    for c in range(3):
        g = feat_ref[c]
        mu = jnp.mean(g, axis=1, keepdims=True)
        out_ref[0, c] = (g - mu) * 10.0
    for c in range(3, 6):
        out_ref[0, c] = feat_ref[c]


@jax.jit
def _knn_feature(x):
    b, c, n = x.shape
    x3t = jnp.transpose(x[:, :3, :], (0, 2, 1))
    xp = jnp.transpose(
        x.reshape(b, 6, _HI, _LANES), (0, 2, 1, 3)
    ).reshape(b, _HI, 6 * _LANES)
    grid = (b, n // _R)
    return pl.pallas_call(
        _knn_feature_kernel,
        grid=grid,
        in_specs=[
            pl.BlockSpec((1, 6, n), lambda b_, t: (b_, 0, 0)),
            pl.BlockSpec((1, _R, 3), lambda b_, t: (b_, t, 0)),
            pl.BlockSpec((1, _HI, 6 * _LANES), lambda b_, t: (b_, 0, 0)),
        ],
        out_specs=pl.BlockSpec((1, 6, _R, _K), lambda b_, t: (b_, 0, t, 0)),
        out_shape=jax.ShapeDtypeStruct((b, 6, n, _K), jnp.float32),
        scratch_shapes=[
            pltpu.VMEM((_R, _N), jnp.float32),
            pltpu.VMEM((6, _R, _K), jnp.float32),
        ],
        compiler_params=pltpu.CompilerParams(
            dimension_semantics=("parallel", "parallel"),
        ),
    )(x, x3t, xp)


def kernel(x, k):
    del k
    return _knn_feature(x)


# speedup vs baseline: 1.7779x; 1.7779x over previous
import functools

import jax
import jax.numpy as jnp
from jax import lax
from jax.experimental import pallas as pl
from jax.experimental.pallas import tpu as pltpu

_N = 8192
_K = 20
_R = 256
_LANES = 128
_HI = _N // _LANES


def _knn_feature_kernel(x_ref, x3t_ref, xp_ref, out_ref, d_ref, feat_ref):
    q = x3t_ref[0]
    keys = x_ref[0, :3, :]
    inner = jnp.dot(q, keys, preferred_element_type=jnp.float32)
    inner_m2 = -2.0 * inner
    xxq = jnp.sum(q * q, axis=1, keepdims=True)
    xxk = jnp.sum(keys * keys, axis=0, keepdims=True)
    d_ref[...] = ((-xxq) - inner_m2) - xxk

    iota = lax.broadcasted_iota(jnp.int32, (_R, _N), 1)
    iota_hi = lax.broadcasted_iota(jnp.int32, (_R, _HI), 1)
    iota_lo = lax.broadcasted_iota(jnp.int32, (_R, _LANES), 1)
    xp = xp_ref[0]

    for j in range(_K):
        d = d_ref[...]
        m = jnp.max(d, axis=1, keepdims=True)
        cand = jnp.where(d == m, iota, jnp.int32(_N))
        fi = jnp.min(cand, axis=1, keepdims=True)
        if j + 1 < _K:
            d_ref[...] = jnp.where(iota == fi, jnp.float32(-1e30), d)
        hi = fi // _LANES
        lo = fi - hi * _LANES
        ohhi = (iota_hi == hi).astype(jnp.float32)
        t2 = jax.lax.dot(ohhi, xp, precision=jax.lax.Precision.HIGHEST,
                         preferred_element_type=jnp.float32)
        ohlo = (iota_lo == lo).astype(jnp.float32)
        for c in range(6):
            sl = t2[:, c * _LANES:(c + 1) * _LANES]
            feat_ref[c, :, j] = jnp.sum(sl * ohlo, axis=1)

    for c in range(3):
        g = feat_ref[c]
        mu = jnp.mean(g, axis=1, keepdims=True)
        out_ref[0, c] = (g - mu) * 10.0
    for c in range(3, 6):
        out_ref[0, c] = feat_ref[c]


@jax.jit
def _knn_feature(x):
    b, c, n = x.shape
    x3t = jnp.transpose(x[:, :3, :], (0, 2, 1))
    xp = jnp.transpose(
        x.reshape(b, 6, _HI, _LANES), (0, 2, 1, 3)
    ).reshape(b, _HI, 6 * _LANES)
    grid = (b, n // _R)
    return pl.pallas_call(
        _knn_feature_kernel,
        grid=grid,
        in_specs=[
            pl.BlockSpec((1, 6, n), lambda b_, t: (b_, 0, 0)),
            pl.BlockSpec((1, _R, 3), lambda b_, t: (b_, t, 0)),
            pl.BlockSpec((1, _HI, 6 * _LANES), lambda b_, t: (b_, 0, 0)),
        ],
        out_specs=pl.BlockSpec((1, 6, _R, _K), lambda b_, t: (b_, 0, t, 0)),
        out_shape=jax.ShapeDtypeStruct((b, 6, n, _K), jnp.float32),
        scratch_shapes=[
            pltpu.VMEM((_R, _N), jnp.float32),
            pltpu.VMEM((6, _R, _K), jnp.float32),
        ],
        compiler_params=pltpu.CompilerParams(
            dimension_semantics=("parallel", "parallel"),
        ),
    )(x, x3t, xp)


def kernel(x, k):
    del k
    return _knn_feature(x)


# speedup vs baseline: 2.0844x; 1.1724x over previous
import functools

import jax
import jax.numpy as jnp
from jax import lax
from jax.experimental import pallas as pl
from jax.experimental.pallas import tpu as pltpu

_N = 8192
_K = 20
_R = 256
_LANES = 128
_SLABS = _N // _LANES
_S = 6


def _knn_feature_kernel(x_ref, x3t_ref, xp_ref, out_ref, d_ref, idx_ref,
                        feat_ref):
    q = x3t_ref[0]
    keys = x_ref[0, :3, :]
    inner = jnp.dot(q, keys, preferred_element_type=jnp.float32)
    inner_m2 = -2.0 * inner
    xxq = jnp.sum(q * q, axis=1, keepdims=True)
    xxk = jnp.sum(keys * keys, axis=0, keepdims=True)
    d_ref[...] = ((-xxq) - inner_m2) - xxk

    d0 = d_ref[...]
    slabs = [d0[:, a * _LANES:(a + 1) * _LANES] for a in range(_SLABS)]
    ms, As = [], []
    neg = jnp.float32(-1e30)
    for s in range(_S):
        m = functools.reduce(jnp.maximum, slabs)
        a_s = functools.reduce(
            jnp.minimum,
            [jnp.where(slabs[a] == m, jnp.int32(a), jnp.int32(_SLABS))
             for a in range(_SLABS)])
        ms.append(m)
        As.append(a_s)
        if s + 1 < _S:
            slabs = [jnp.where(a_s == a, neg, slabs[a])
                     for a in range(_SLABS)]

    lane = lax.broadcasted_iota(jnp.int32, (_R, _LANES), 1)
    cnt = jnp.zeros((_R, _LANES), jnp.int32)
    big = jnp.int32(1 << 20)
    for j in range(_K):
        cur = ms[_S - 1]
        acur = As[_S - 1]
        for s in range(_S - 2, -1, -1):
            sel = cnt == s
            cur = jnp.where(sel, ms[s], cur)
            acur = jnp.where(sel, As[s], acur)
        cur = jnp.where(cnt >= _S, neg, cur)
        mB = jnp.max(cur, axis=1, keepdims=True)
        g = acur * _LANES + lane
        fi = jnp.min(jnp.where(cur == mB, g, big), axis=1,
                     keepdims=True)
        cB = lax.rem(fi, jnp.int32(_LANES))
        cnt = cnt + (lane == cB).astype(jnp.int32)
        idx_ref[:, j] = fi[:, 0]

    overflow = jnp.max(cnt) >= _S

    @pl.when(overflow)
    def _fallback():
        iota = lax.broadcasted_iota(jnp.int32, (_R, _N), 1)
        for j in range(_K):
            d = d_ref[...]
            m = jnp.max(d, axis=1, keepdims=True)
            cand = jnp.where(d == m, iota, jnp.int32(_N))
            fi = jnp.min(cand, axis=1, keepdims=True)
            if j + 1 < _K:
                d_ref[...] = jnp.where(iota == fi, neg, d)
            idx_ref[:, j] = fi[:, 0]

    iota_hi = lax.broadcasted_iota(jnp.int32, (_R, _SLABS), 1)
    iota_lo = lax.broadcasted_iota(jnp.int32, (_R, _LANES), 1)
    xp = xp_ref[0]
    for j in range(_K):
        fi = idx_ref[:, j][:, None]
        hi = fi // _LANES
        lo = fi - hi * _LANES
        ohhi = (iota_hi == hi).astype(jnp.float32)
        t2 = jax.lax.dot(ohhi, xp, precision=jax.lax.Precision.HIGHEST,
                         preferred_element_type=jnp.float32)
        ohlo = (iota_lo == lo).astype(jnp.float32)
        for c in range(6):
            sl = t2[:, c * _LANES:(c + 1) * _LANES]
            feat_ref[c, :, j] = jnp.sum(sl * ohlo, axis=1)

    for c in range(3):
        gch = feat_ref[c]
        mu = jnp.mean(gch, axis=1, keepdims=True)
        out_ref[0, c] = (gch - mu) * 10.0
    for c in range(3, 6):
        out_ref[0, c] = feat_ref[c]


@jax.jit
def _knn_feature(x):
    b, c, n = x.shape
    x3t = jnp.transpose(x[:, :3, :], (0, 2, 1))
    xp = jnp.transpose(
        x.reshape(b, 6, _SLABS, _LANES), (0, 2, 1, 3)
    ).reshape(b, _SLABS, 6 * _LANES)
    grid = (b, n // _R)
    return pl.pallas_call(
        _knn_feature_kernel,
        grid=grid,
        in_specs=[
            pl.BlockSpec((1, 6, n), lambda b_, t: (b_, 0, 0)),
            pl.BlockSpec((1, _R, 3), lambda b_, t: (b_, t, 0)),
            pl.BlockSpec((1, _SLABS, 6 * _LANES), lambda b_, t: (b_, 0, 0)),
        ],
        out_specs=pl.BlockSpec((1, 6, _R, _K), lambda b_, t: (b_, 0, t, 0)),
        out_shape=jax.ShapeDtypeStruct((b, 6, n, _K), jnp.float32),
        scratch_shapes=[
            pltpu.VMEM((_R, _N), jnp.float32),
            pltpu.VMEM((_R, _K), jnp.int32),
            pltpu.VMEM((6, _R, _K), jnp.float32),
        ],
        compiler_params=pltpu.CompilerParams(
            dimension_semantics=("parallel", "parallel"),
        ),
    )(x, x3t, xp)


def kernel(x, k):
    del k
    return _knn_feature(x)
